# baseline (speedup 1.0000x reference)
import jax
import jax.numpy as jnp
from jax import lax
from jax.experimental import pallas as pl
from jax.experimental.pallas import tpu as pltpu

N_ROWS = 100000
N_COLS = 128

B_TC = 200
K_TC = 20
ROUNDS_TC = 25
assert K_TC * B_TC * ROUNDS_TC == N_ROWS


def _tc_body(x_hbm, o_ref, bufs, acc_ref, sems):
    def chunk(i):
        return x_hbm.at[pl.ds(i * B_TC, B_TC), :]

    acc_ref[...] = jnp.zeros_like(acc_ref)
    for j in range(K_TC):
        pltpu.async_copy(chunk(j), bufs.at[j], sems.at[j])

    def round_body(r, _):
        for j in range(K_TC):
            pltpu.make_async_copy(chunk(r * K_TC + j), bufs.at[j],
                                  sems.at[j]).wait()
            acc_ref[...] += jnp.sum(
                bufs[j].reshape(B_TC // 8, 8, N_COLS), axis=0)

            @pl.when(r < ROUNDS_TC - 1)
            def _():
                pltpu.async_copy(chunk((r + 1) * K_TC + j), bufs.at[j],
                                 sems.at[j])
        return 0

    lax.fori_loop(0, ROUNDS_TC, round_body, 0)
    o_ref[...] = jnp.sum(acc_ref[...], axis=0, keepdims=True)


_tc_call = pl.pallas_call(
    _tc_body,
    in_specs=[pl.BlockSpec(memory_space=pl.ANY)],
    out_shape=jax.ShapeDtypeStruct((1, N_COLS), jnp.float32),
    scratch_shapes=[
        pltpu.VMEM((K_TC, B_TC, N_COLS), jnp.float32),
        pltpu.VMEM((8, N_COLS), jnp.float32),
        pltpu.SemaphoreType.DMA((K_TC,)),
    ],
)


def kernel(x):
    return _tc_call(x)


# speedup vs baseline: 1.0355x; 1.0355x over previous
import jax
import jax.numpy as jnp
from jax.experimental import pallas as pl
from jax.experimental.pallas import tpu as pltpu

N_ROWS = 100000
N_COLS = 128

B_TC = 5000
NSTREAM = 1
G_TC = 20
assert NSTREAM * B_TC * G_TC == N_ROWS


def _tc_body(*refs):
    x_refs = refs[:NSTREAM]
    o_ref = refs[NSTREAM]
    acc_ref = refs[NSTREAM + 1]
    i = pl.program_id(0)

    @pl.when(i == 0)
    def _():
        acc_ref[...] = jnp.zeros_like(acc_ref)

    part = acc_ref[...]
    for x_ref in x_refs:
        part += jnp.sum(x_ref[...].reshape(B_TC // 8, 8, N_COLS), axis=0)
    acc_ref[...] = part

    @pl.when(i == G_TC - 1)
    def _():
        o_ref[...] = jnp.sum(acc_ref[...], axis=0, keepdims=True)


_tc_call = pl.pallas_call(
    _tc_body,
    grid=(G_TC,),
    in_specs=[
        pl.BlockSpec((B_TC, N_COLS), lambda i, _k=k: (i * NSTREAM + _k, 0))
        for k in range(NSTREAM)
    ],
    out_specs=pl.BlockSpec((1, N_COLS), lambda i: (0, 0)),
    out_shape=jax.ShapeDtypeStruct((1, N_COLS), jnp.float32),
    scratch_shapes=[pltpu.VMEM((8, N_COLS), jnp.float32)],
)


def kernel(x):
    return _tc_call(*([x] * NSTREAM))


# speedup vs baseline: 1.2530x; 1.2100x over previous
import jax
import jax.numpy as jnp
from jax.experimental import pallas as pl
from jax.experimental.pallas import tpu as pltpu

N_ROWS = 100000
N_COLS = 128

B_TC = 10000
NSTREAM = 1
G_TC = 10
assert NSTREAM * B_TC * G_TC == N_ROWS


def _tc_body(*refs):
    x_refs = refs[:NSTREAM]
    o_ref = refs[NSTREAM]
    acc_ref = refs[NSTREAM + 1]
    i = pl.program_id(0)

    @pl.when(i == 0)
    def _():
        acc_ref[...] = jnp.zeros_like(acc_ref)

    part = acc_ref[...]
    for x_ref in x_refs:
        part += jnp.sum(x_ref[...].reshape(B_TC // 8, 8, N_COLS), axis=0)
    acc_ref[...] = part

    @pl.when(i == G_TC - 1)
    def _():
        o_ref[...] = jnp.sum(acc_ref[...], axis=0, keepdims=True)


_tc_call = pl.pallas_call(
    _tc_body,
    grid=(G_TC,),
    in_specs=[
        pl.BlockSpec((B_TC, N_COLS), lambda i, _k=k: (i * NSTREAM + _k, 0))
        for k in range(NSTREAM)
    ],
    out_specs=pl.BlockSpec((1, N_COLS), lambda i: (0, 0)),
    out_shape=jax.ShapeDtypeStruct((1, N_COLS), jnp.float32),
    scratch_shapes=[pltpu.VMEM((8, N_COLS), jnp.float32)],
)


def kernel(x):
    return _tc_call(*([x] * NSTREAM))


# speedup vs baseline: 1.5151x; 1.2092x over previous
import jax
import jax.numpy as jnp
from jax.experimental import pallas as pl
from jax.experimental.pallas import tpu as pltpu

N_ROWS = 100000
N_COLS = 128

B_TC = 10000
NSTREAM = 2
G_TC = 5
assert NSTREAM * B_TC * G_TC == N_ROWS


def _tc_body(*refs):
    x_refs = refs[:NSTREAM]
    o_ref = refs[NSTREAM]
    acc_ref = refs[NSTREAM + 1]
    i = pl.program_id(0)

    @pl.when(i == 0)
    def _():
        acc_ref[...] = jnp.zeros_like(acc_ref)

    part = acc_ref[...]
    for x_ref in x_refs:
        part += jnp.sum(x_ref[...].reshape(B_TC // 8, 8, N_COLS), axis=0)
    acc_ref[...] = part

    @pl.when(i == G_TC - 1)
    def _():
        o_ref[...] = jnp.sum(acc_ref[...], axis=0, keepdims=True)


_tc_call = pl.pallas_call(
    _tc_body,
    grid=(G_TC,),
    in_specs=[
        pl.BlockSpec((B_TC, N_COLS), lambda i, _k=k: (i * NSTREAM + _k, 0))
        for k in range(NSTREAM)
    ],
    out_specs=pl.BlockSpec((1, N_COLS), lambda i: (0, 0)),
    out_shape=jax.ShapeDtypeStruct((1, N_COLS), jnp.float32),
    scratch_shapes=[pltpu.VMEM((8, N_COLS), jnp.float32)],
)


def kernel(x):
    return _tc_call(*([x] * NSTREAM))


# speedup vs baseline: 1.5507x; 1.0235x over previous
import jax
import jax.numpy as jnp
from jax.experimental import pallas as pl
from jax.experimental.pallas import tpu as pltpu

N_ROWS = 100000
N_COLS = 128

B_TC = 5000
NSTREAM = 4
G_TC = 5
assert NSTREAM * B_TC * G_TC == N_ROWS


def _tc_body(*refs):
    x_refs = refs[:NSTREAM]
    o_ref = refs[NSTREAM]
    acc_ref = refs[NSTREAM + 1]
    i = pl.program_id(0)

    @pl.when(i == 0)
    def _():
        acc_ref[...] = jnp.zeros_like(acc_ref)

    part = acc_ref[...]
    for x_ref in x_refs:
        part += jnp.sum(x_ref[...].reshape(B_TC // 8, 8, N_COLS), axis=0)
    acc_ref[...] = part

    @pl.when(i == G_TC - 1)
    def _():
        o_ref[...] = jnp.sum(acc_ref[...], axis=0, keepdims=True)


_tc_call = pl.pallas_call(
    _tc_body,
    grid=(G_TC,),
    in_specs=[
        pl.BlockSpec((B_TC, N_COLS), lambda i, _k=k: (i * NSTREAM + _k, 0))
        for k in range(NSTREAM)
    ],
    out_specs=pl.BlockSpec((1, N_COLS), lambda i: (0, 0)),
    out_shape=jax.ShapeDtypeStruct((1, N_COLS), jnp.float32),
    scratch_shapes=[pltpu.VMEM((8, N_COLS), jnp.float32)],
)


def kernel(x):
    return _tc_call(*([x] * NSTREAM))


# speedup vs baseline: 1.5779x; 1.0176x over previous
import jax
import jax.numpy as jnp
from jax.experimental import pallas as pl
from jax.experimental.pallas import tpu as pltpu

N_ROWS = 100000
N_COLS = 128

B_TC = 5000
NSTREAM = 5
G_TC = 4
assert NSTREAM * B_TC * G_TC == N_ROWS


def _tc_body(*refs):
    x_refs = refs[:NSTREAM]
    o_ref = refs[NSTREAM]
    acc_ref = refs[NSTREAM + 1]
    i = pl.program_id(0)

    @pl.when(i == 0)
    def _():
        acc_ref[...] = jnp.zeros_like(acc_ref)

    part = acc_ref[...]
    for x_ref in x_refs:
        part += jnp.sum(x_ref[...].reshape(B_TC // 8, 8, N_COLS), axis=0)
    acc_ref[...] = part

    @pl.when(i == G_TC - 1)
    def _():
        o_ref[...] = jnp.sum(acc_ref[...], axis=0, keepdims=True)


_tc_call = pl.pallas_call(
    _tc_body,
    grid=(G_TC,),
    in_specs=[
        pl.BlockSpec((B_TC, N_COLS), lambda i, _k=k: (i * NSTREAM + _k, 0))
        for k in range(NSTREAM)
    ],
    out_specs=pl.BlockSpec((1, N_COLS), lambda i: (0, 0)),
    out_shape=jax.ShapeDtypeStruct((1, N_COLS), jnp.float32),
    scratch_shapes=[pltpu.VMEM((8, N_COLS), jnp.float32)],
)


def kernel(x):
    return _tc_call(*([x] * NSTREAM))
